# baseline (speedup 1.0000x reference)
import functools

import jax
import jax.numpy as jnp
from jax import lax
from jax.experimental import pallas as pl
from jax.experimental.pallas import tpu as pltpu
from jax.experimental.pallas import tpu_sc as plsc

N = 10000
E = 320000
D = 128
H = 128
C = 16

NS = 16
CH = 128
MBLK = 16
NCHUNK_TOT = 2560
NCHUNK_W = NCHUNK_TOT // NS
E_PAD = NCHUNK_TOT * CH
N_PAD = 10240
RPT = N_PAD // NS

_f32 = jnp.float32


def _make_edge_scatter(feat, gather):
    rows_shape = (CH,) + feat
    feat_elems = 1
    for f in feat:
        feat_elems *= f
    nvec = (CH * feat_elems) // 16

    mesh = plsc.VectorSubcoreMesh(core_axis_name="c", subcore_axis_name="s",
                                  num_cores=1)

    def body(*refs):
        if gather:
            (hs_hbm, zeros_hbm, src_hbm, dst_hbm, out_hbm, src_v, dst_v,
             rows_v, acc_sh, gsem0, gsem1, ssem0, ssem1) = refs
        else:
            zeros_hbm, dst_hbm, out_hbm, dst_v, rows_v, acc_sh = refs
        s = lax.axis_index("s")

        r0 = s * RPT
        pltpu.sync_copy(zeros_hbm.at[pl.ds(r0, RPT)],
                        acc_sh.at[pl.ds(r0, RPT)])
        plsc.subcore_barrier()

        if gather:
            gsems = [gsem0, gsem1]
            ssems = [ssem0, ssem1]

            def g_start(j, buf):
                pltpu.async_copy(hs_hbm.at[src_v.at[j]], rows_v.at[buf],
                                 gsems[buf])

            def g_wait(j, buf):
                pltpu.make_async_copy(hs_hbm.at[src_v.at[j]],
                                      rows_v.at[buf], gsems[buf]).wait()

            def s_start(j, buf):
                pltpu.async_copy(rows_v.at[buf], acc_sh.at[dst_v.at[j]],
                                 ssems[buf], add=True)

            def s_wait(j, buf):
                pltpu.make_async_copy(rows_v.at[buf], acc_sh.at[dst_v.at[j]],
                                      ssems[buf]).wait()

            def block(b, _):
                cs = s * NCHUNK_W + b * MBLK
                pltpu.sync_copy(src_hbm.at[pl.ds(cs, MBLK)], src_v)
                pltpu.sync_copy(dst_hbm.at[pl.ds(cs, MBLK)], dst_v)
                g_start(0, 0)
                g_start(1, 1)

                def chunk(it, _):
                    j = 2 * it
                    for buf in range(2):
                        g_wait(j + buf, buf)
                        s_start(j + buf, buf)
                    for buf in range(2):
                        s_wait(j + buf, buf)

                        @pl.when(j + buf + 2 < MBLK)
                        def _():
                            g_start(j + buf + 2, buf)

                    return 0

                lax.fori_loop(0, MBLK // 2, chunk, 0)
                return 0

            lax.fori_loop(0, NCHUNK_W // MBLK, block, 0)
        else:
            pltpu.sync_copy(dst_hbm.at[pl.ds(s * NCHUNK_W, NCHUNK_W)], dst_v)

            def fbody(t, _):
                rows_v[pl.ds(t * 16, 16)] = jnp.full((16,), 1.0, _f32)
                return 0

            lax.fori_loop(0, nvec, fbody, 0)

            def chunk(j, _):
                pltpu.sync_copy(rows_v, acc_sh.at[dst_v.at[j]], add=True)
                return 0

            lax.fori_loop(0, NCHUNK_W, chunk, 0)
        plsc.subcore_barrier()

        pltpu.sync_copy(acc_sh.at[pl.ds(r0, RPT)],
                        out_hbm.at[pl.ds(r0, RPT)])

    scratch = []
    if gather:
        scratch.append(pltpu.VMEM((MBLK, CH), jnp.int32))
    scratch += [
        pltpu.VMEM((MBLK if gather else NCHUNK_W, CH), jnp.int32),
        pltpu.VMEM(((2,) if gather else ()) + rows_shape, _f32),
        pltpu.VMEM_SHARED((N_PAD,) + feat, _f32),
    ]
    if gather:
        scratch += [pltpu.SemaphoreType.DMA] * 4

    kern = pl.kernel(
        body,
        out_type=jax.ShapeDtypeStruct((N_PAD,) + feat, _f32),
        mesh=mesh,
        scratch_types=scratch,
    )
    return kern


_deg_scatter = _make_edge_scatter((), gather=False)
_scatter_h = _make_edge_scatter((H,), gather=True)


def _tc_first(x, w, degp):
    def body(x_ref, w_ref, deg_ref, hs_ref, dis_ref):
        dis = lax.rsqrt(deg_ref[:N, :] + 1.0)
        dis_ref[...] = dis
        hs_ref[...] = dis * jnp.dot(x_ref[...], w_ref[...],
                                    preferred_element_type=_f32)

    return pl.pallas_call(
        body,
        out_shape=(
            jax.ShapeDtypeStruct((N, w.shape[1]), _f32),
            jax.ShapeDtypeStruct((N, 1), _f32),
        ),
    )(x, w, degp)


def _tc_mid(p, hs, dis, b, w):
    def body(p_ref, hs_ref, dis_ref, b_ref, w_ref, out_ref):
        dis = dis_ref[...]
        a = dis * (p_ref[:N, :] + hs_ref[...]) + b_ref[...]
        h = jnp.maximum(a, 0.0)
        out_ref[...] = dis * jnp.dot(h, w_ref[...], preferred_element_type=_f32)

    return pl.pallas_call(
        body,
        out_shape=jax.ShapeDtypeStruct((N, w.shape[1]), _f32),
    )(p, hs, dis, b, w)


def _tc_last(p, hs, dis, b):
    def body(p_ref, hs_ref, dis_ref, b_ref, out_ref):
        a = dis_ref[...] * (p_ref[:N, :C] + hs_ref[:, :C]) + b_ref[...]
        m = jnp.max(a, axis=1, keepdims=True)
        lse = m + jnp.log(jnp.sum(jnp.exp(a - m), axis=1, keepdims=True))
        out_ref[...] = a - lse

    return pl.pallas_call(
        body,
        out_shape=jax.ShapeDtypeStruct((N, C), _f32),
    )(p, hs, dis, b)


@jax.jit
def kernel(x, edge_index, W1, b1, W2, b2, W3, b3, W4, b4):
    pad = E_PAD - E
    src = jnp.concatenate(
        [edge_index[0], jnp.zeros((pad,), jnp.int32)]).reshape(NCHUNK_TOT, CH)
    dst = jnp.concatenate(
        [edge_index[1], jnp.full((pad,), N, jnp.int32)]).reshape(NCHUNK_TOT, CH)

    z1 = jnp.zeros((N_PAD,), _f32)
    zH = jnp.zeros((N_PAD, H), _f32)

    degp = _deg_scatter(z1, dst).reshape(N_PAD, 1)
    hs1, dis = _tc_first(x, W1, degp)

    p = _scatter_h(hs1, zH, src, dst)
    hs2 = _tc_mid(p, hs1, dis, b1.reshape(1, H), W2)
    p = _scatter_h(hs2, zH, src, dst)
    hs3 = _tc_mid(p, hs2, dis, b2.reshape(1, H), W2)
    p = _scatter_h(hs3, zH, src, dst)
    hs4 = _tc_mid(p, hs3, dis, b2.reshape(1, H), W3)
    p = _scatter_h(hs4, zH, src, dst)
    W4p = jnp.pad(W4, ((0, 0), (0, H - C)))
    hs5 = _tc_mid(p, hs4, dis, b3.reshape(1, H), W4p)
    p = _scatter_h(hs5, zH, src, dst)
    return _tc_last(p, hs5, dis, b4.reshape(1, C))


# speedup vs baseline: 1.0007x; 1.0007x over previous
import functools

import jax
import jax.numpy as jnp
from jax import lax
from jax.experimental import pallas as pl
from jax.experimental.pallas import tpu as pltpu
from jax.experimental.pallas import tpu_sc as plsc

N = 10000
E = 320000
D = 128
H = 128
C = 16

NS = 16
CH = 128
MBLK = 16
NCHUNK_TOT = 2560
NCHUNK_W = NCHUNK_TOT // NS
E_PAD = NCHUNK_TOT * CH
N_PAD = 10240
RPT = N_PAD // NS

_f32 = jnp.float32


def _make_edge_scatter(feat, gather):
    rows_shape = (CH,) + feat
    feat_elems = 1
    for f in feat:
        feat_elems *= f
    nvec = (CH * feat_elems) // 16

    mesh = plsc.VectorSubcoreMesh(core_axis_name="c", subcore_axis_name="s")

    def body(*refs):
        if gather:
            (hs_hbm, zeros_hbm, src_hbm, dst_hbm, out_hbm, src_v, dst_v,
             rows_v, acc_sh, gsem0, gsem1, ssem0, ssem1) = refs
        else:
            zeros_hbm, dst_hbm, out_hbm, dst_v, rows_v, acc_sh = refs
        c = lax.axis_index("c")
        s = lax.axis_index("s")
        on = c == 0

        r0 = s * RPT

        @pl.when(on)
        def _():
            pltpu.sync_copy(zeros_hbm.at[pl.ds(r0, RPT)],
                            acc_sh.at[pl.ds(r0, RPT)])

        plsc.subcore_barrier()

        if gather:
            gsems = [gsem0, gsem1]
            ssems = [ssem0, ssem1]

            def g_start(j, buf):
                pltpu.async_copy(hs_hbm.at[src_v.at[j]], rows_v.at[buf],
                                 gsems[buf])

            def g_wait(j, buf):
                pltpu.make_async_copy(hs_hbm.at[src_v.at[j]],
                                      rows_v.at[buf], gsems[buf]).wait()

            def s_start(j, buf):
                pltpu.async_copy(rows_v.at[buf], acc_sh.at[dst_v.at[j]],
                                 ssems[buf], add=True)

            def s_wait(j, buf):
                pltpu.make_async_copy(rows_v.at[buf], acc_sh.at[dst_v.at[j]],
                                      ssems[buf]).wait()

            def block(b, _):
                cs = s * NCHUNK_W + b * MBLK
                pltpu.sync_copy(src_hbm.at[pl.ds(cs, MBLK)], src_v)
                pltpu.sync_copy(dst_hbm.at[pl.ds(cs, MBLK)], dst_v)
                g_start(0, 0)
                g_start(1, 1)

                def chunk(it, _):
                    j = 2 * it
                    for buf in range(2):
                        g_wait(j + buf, buf)
                        s_start(j + buf, buf)
                    for buf in range(2):
                        s_wait(j + buf, buf)

                        @pl.when(j + buf + 2 < MBLK)
                        def _():
                            g_start(j + buf + 2, buf)

                    return 0

                lax.fori_loop(0, MBLK // 2, chunk, 0)
                return 0

            @pl.when(on)
            def _():
                lax.fori_loop(0, NCHUNK_W // MBLK, block, 0)
        else:
            @pl.when(on)
            def _():
                pltpu.sync_copy(dst_hbm.at[pl.ds(s * NCHUNK_W, NCHUNK_W)],
                                dst_v)

                def fbody(t, _):
                    rows_v[pl.ds(t * 16, 16)] = jnp.full((16,), 1.0, _f32)
                    return 0

                lax.fori_loop(0, nvec, fbody, 0)

                def chunk(j, _):
                    pltpu.sync_copy(rows_v, acc_sh.at[dst_v.at[j]], add=True)
                    return 0

                lax.fori_loop(0, NCHUNK_W, chunk, 0)
        plsc.subcore_barrier()

        @pl.when(on)
        def _():
            pltpu.sync_copy(acc_sh.at[pl.ds(r0, RPT)],
                            out_hbm.at[pl.ds(r0, RPT)])

    scratch = []
    if gather:
        scratch.append(pltpu.VMEM((MBLK, CH), jnp.int32))
    scratch += [
        pltpu.VMEM((MBLK if gather else NCHUNK_W, CH), jnp.int32),
        pltpu.VMEM(((2,) if gather else ()) + rows_shape, _f32),
        pltpu.VMEM_SHARED((N_PAD,) + feat, _f32),
    ]
    if gather:
        scratch += [pltpu.SemaphoreType.DMA] * 4

    kern = pl.kernel(
        body,
        out_type=jax.ShapeDtypeStruct((N_PAD,) + feat, _f32),
        mesh=mesh,
        scratch_types=scratch,
    )
    return kern


_deg_scatter = _make_edge_scatter((), gather=False)
_scatter_h = _make_edge_scatter((H,), gather=True)


def _tc_first(x, w, degp):
    def body(x_ref, w_ref, deg_ref, hs_ref, dis_ref):
        dis = lax.rsqrt(deg_ref[:N, :] + 1.0)
        dis_ref[...] = dis
        hs_ref[...] = dis * jnp.dot(x_ref[...], w_ref[...],
                                    preferred_element_type=_f32)

    return pl.pallas_call(
        body,
        out_shape=(
            jax.ShapeDtypeStruct((N, w.shape[1]), _f32),
            jax.ShapeDtypeStruct((N, 1), _f32),
        ),
    )(x, w, degp)


def _tc_mid(p, hs, dis, b, w):
    def body(p_ref, hs_ref, dis_ref, b_ref, w_ref, out_ref):
        dis = dis_ref[...]
        a = dis * (p_ref[:N, :] + hs_ref[...]) + b_ref[...]
        h = jnp.maximum(a, 0.0)
        out_ref[...] = dis * jnp.dot(h, w_ref[...], preferred_element_type=_f32)

    return pl.pallas_call(
        body,
        out_shape=jax.ShapeDtypeStruct((N, w.shape[1]), _f32),
    )(p, hs, dis, b, w)


def _tc_last(p, hs, dis, b):
    def body(p_ref, hs_ref, dis_ref, b_ref, out_ref):
        a = dis_ref[...] * (p_ref[:N, :C] + hs_ref[:, :C]) + b_ref[...]
        m = jnp.max(a, axis=1, keepdims=True)
        lse = m + jnp.log(jnp.sum(jnp.exp(a - m), axis=1, keepdims=True))
        out_ref[...] = a - lse

    return pl.pallas_call(
        body,
        out_shape=jax.ShapeDtypeStruct((N, C), _f32),
    )(p, hs, dis, b)


@jax.jit
def kernel(x, edge_index, W1, b1, W2, b2, W3, b3, W4, b4):
    pad = E_PAD - E
    src = jnp.concatenate(
        [edge_index[0], jnp.zeros((pad,), jnp.int32)]).reshape(NCHUNK_TOT, CH)
    dst = jnp.concatenate(
        [edge_index[1], jnp.full((pad,), N, jnp.int32)]).reshape(NCHUNK_TOT, CH)

    z1 = jnp.zeros((N_PAD,), _f32)
    zH = jnp.zeros((N_PAD, H), _f32)

    degp = _deg_scatter(z1, dst).reshape(N_PAD, 1)
    hs1, dis = _tc_first(x, W1, degp)

    p = _scatter_h(hs1, zH, src, dst)
    hs2 = _tc_mid(p, hs1, dis, b1.reshape(1, H), W2)
    p = _scatter_h(hs2, zH, src, dst)
    hs3 = _tc_mid(p, hs2, dis, b2.reshape(1, H), W2)
    p = _scatter_h(hs3, zH, src, dst)
    hs4 = _tc_mid(p, hs3, dis, b2.reshape(1, H), W3)
    p = _scatter_h(hs4, zH, src, dst)
    W4p = jnp.pad(W4, ((0, 0), (0, H - C)))
    hs5 = _tc_mid(p, hs4, dis, b3.reshape(1, H), W4p)
    p = _scatter_h(hs5, zH, src, dst)
    return _tc_last(p, hs5, dis, b4.reshape(1, C))


# speedup vs baseline: 2.0231x; 2.0217x over previous
import functools

import jax
import jax.numpy as jnp
from jax import lax
from jax.experimental import pallas as pl
from jax.experimental.pallas import tpu as pltpu
from jax.experimental.pallas import tpu_sc as plsc

N = 10000
E = 320000
D = 128
H = 128
C = 16

NS = 16
CH = 128
MBLK = 16
NCHUNK_TOT = 2560
NCHUNK_W = NCHUNK_TOT // NS
E_PAD = NCHUNK_TOT * CH
N_PAD = 10240
RPT = N_PAD // NS

_f32 = jnp.float32


def _make_edge_scatter(feat, gather):
    rows_shape = (CH,) + feat
    feat_elems = 1
    for f in feat:
        feat_elems *= f
    nvec = (CH * feat_elems) // 16

    mesh = plsc.VectorSubcoreMesh(core_axis_name="c", subcore_axis_name="s")

    def body(*refs):
        if gather:
            (hs_hbm, zeros_hbm, src_hbm, dst_hbm, out_hbm, src_v, dst_v,
             rows_v, acc_sh, gsem0, gsem1, ssem0, ssem1) = refs
        else:
            zeros_hbm, dst_hbm, out_hbm, dst_v, rows_v, acc_sh = refs
        c = lax.axis_index("c")
        s = lax.axis_index("s")
        on = c == 0

        r0 = s * RPT

        @pl.when(on)
        def _():
            pltpu.sync_copy(zeros_hbm.at[pl.ds(r0, RPT)],
                            acc_sh.at[pl.ds(r0, RPT)])

        plsc.subcore_barrier()

        if gather:
            gsems = [gsem0, gsem1]
            ssems = [ssem0, ssem1]

            def g_start(j, buf):
                pltpu.async_copy(hs_hbm.at[src_v.at[j]], rows_v.at[buf],
                                 gsems[buf])

            def g_wait(j, buf):
                pltpu.make_async_copy(hs_hbm.at[src_v.at[j]],
                                      rows_v.at[buf], gsems[buf]).wait()

            def s_start(j, buf):
                pltpu.async_copy(rows_v.at[buf], acc_sh.at[dst_v.at[j]],
                                 ssems[buf], add=True)

            def s_wait(j, buf):
                pltpu.make_async_copy(rows_v.at[buf], acc_sh.at[dst_v.at[j]],
                                      ssems[buf]).wait()

            def block(b, _):
                cs = s * NCHUNK_W + b * MBLK
                pltpu.sync_copy(src_hbm.at[pl.ds(cs, MBLK)], src_v)
                pltpu.sync_copy(dst_hbm.at[pl.ds(cs, MBLK)], dst_v)
                g_start(0, 0)
                g_start(1, 1)

                def chunk(it, _):
                    j = 2 * it
                    for buf in range(2):
                        g_wait(j + buf, buf)
                        s_start(j + buf, buf)
                    for buf in range(2):
                        s_wait(j + buf, buf)

                        @pl.when(j + buf + 2 < MBLK)
                        def _():
                            g_start(j + buf + 2, buf)

                    return 0

                lax.fori_loop(0, MBLK // 2, chunk, 0)
                return 0

            @pl.when(on)
            def _():
                lax.fori_loop(0, NCHUNK_W // MBLK, block, 0)
        else:
            @pl.when(on)
            def _():
                pltpu.sync_copy(dst_hbm.at[pl.ds(s * NCHUNK_W, NCHUNK_W)],
                                dst_v)

                def fbody(t, _):
                    rows_v[pl.ds(t * 16, 16)] = jnp.full((16,), 1.0, _f32)
                    return 0

                lax.fori_loop(0, nvec, fbody, 0)

                def chunk(j, _):
                    pltpu.sync_copy(rows_v, acc_sh.at[dst_v.at[j]], add=True)
                    return 0

                lax.fori_loop(0, NCHUNK_W, chunk, 0)
        plsc.subcore_barrier()

        @pl.when(on)
        def _():
            pltpu.sync_copy(acc_sh.at[pl.ds(r0, RPT)],
                            out_hbm.at[pl.ds(r0, RPT)])

    scratch = []
    if gather:
        scratch.append(pltpu.VMEM((MBLK, CH), jnp.int32))
    scratch += [
        pltpu.VMEM((MBLK if gather else NCHUNK_W, CH), jnp.int32),
        pltpu.VMEM(((2,) if gather else ()) + rows_shape, _f32),
        pltpu.VMEM_SHARED((N_PAD,) + feat, _f32),
    ]
    if gather:
        scratch += [pltpu.SemaphoreType.DMA] * 4

    kern = pl.kernel(
        body,
        out_type=jax.ShapeDtypeStruct((N_PAD,) + feat, _f32),
        mesh=mesh,
        scratch_types=scratch,
    )
    return kern


_deg_scatter = _make_edge_scatter((), gather=False)
_scatter_h = _make_edge_scatter((H,), gather=True)


def _tc_first(x, w, degp):
    def body(x_ref, w_ref, deg_ref, hs_ref, dis_ref):
        dis = lax.rsqrt(deg_ref[:N, :] + 1.0)
        dis_ref[...] = dis
        hs_ref[...] = dis * jnp.dot(x_ref[...], w_ref[...],
                                    preferred_element_type=_f32)

    return pl.pallas_call(
        body,
        out_shape=(
            jax.ShapeDtypeStruct((N, w.shape[1]), _f32),
            jax.ShapeDtypeStruct((N, 1), _f32),
        ),
    )(x, w, degp)


def _tc_mid(p, hs, dis, b, w):
    def body(p_ref, hs_ref, dis_ref, b_ref, w_ref, out_ref):
        dis = dis_ref[...]
        a = dis * (p_ref[:N, :] + hs_ref[...]) + b_ref[...]
        h = jnp.maximum(a, 0.0)
        out_ref[...] = dis * jnp.dot(h, w_ref[...], preferred_element_type=_f32)

    return pl.pallas_call(
        body,
        out_shape=jax.ShapeDtypeStruct((N, w.shape[1]), _f32),
    )(p, hs, dis, b, w)


def _tc_last(p, hs, dis, b):
    def body(p_ref, hs_ref, dis_ref, b_ref, out_ref):
        a = dis_ref[...] * (p_ref[:N, :C] + hs_ref[:, :C]) + b_ref[...]
        m = jnp.max(a, axis=1, keepdims=True)
        lse = m + jnp.log(jnp.sum(jnp.exp(a - m), axis=1, keepdims=True))
        out_ref[...] = a - lse

    return pl.pallas_call(
        body,
        out_shape=jax.ShapeDtypeStruct((N, C), _f32),
    )(p, hs, dis, b)


@jax.jit
def kernel(x, edge_index, W1, b1, W2, b2, W3, b3, W4, b4):
    pad = E_PAD - E
    padi = jnp.arange(pad, dtype=jnp.int32)
    src = jnp.concatenate(
        [edge_index[0], padi % CH]).reshape(NCHUNK_TOT, CH)
    dst = jnp.concatenate(
        [edge_index[1], N + padi % (N_PAD - N)]).reshape(NCHUNK_TOT, CH)

    z1 = jnp.zeros((N_PAD,), _f32)
    zH = jnp.zeros((N_PAD, H), _f32)

    degp = _deg_scatter(z1, dst).reshape(N_PAD, 1)
    hs1, dis = _tc_first(x, W1, degp)

    p = _scatter_h(hs1, zH, src, dst)
    hs2 = _tc_mid(p, hs1, dis, b1.reshape(1, H), W2)
    p = _scatter_h(hs2, zH, src, dst)
    hs3 = _tc_mid(p, hs2, dis, b2.reshape(1, H), W2)
    p = _scatter_h(hs3, zH, src, dst)
    hs4 = _tc_mid(p, hs3, dis, b2.reshape(1, H), W3)
    p = _scatter_h(hs4, zH, src, dst)
    W4p = jnp.pad(W4, ((0, 0), (0, H - C)))
    hs5 = _tc_mid(p, hs4, dis, b3.reshape(1, H), W4p)
    p = _scatter_h(hs5, zH, src, dst)
    return _tc_last(p, hs5, dis, b4.reshape(1, C))


# speedup vs baseline: 2.0777x; 1.0270x over previous
import functools

import jax
import jax.numpy as jnp
from jax import lax
from jax.experimental import pallas as pl
from jax.experimental.pallas import tpu as pltpu
from jax.experimental.pallas import tpu_sc as plsc

N = 10000
E = 320000
D = 128
H = 128
C = 16

NS = 16
CH = 128
MBLK = 40
NCHUNK_TOT = 2560
NCHUNK_W = NCHUNK_TOT // NS
E_PAD = NCHUNK_TOT * CH
N_PAD = 10240
RPT = N_PAD // NS

_f32 = jnp.float32


def _make_edge_scatter(feat, gather):
    rows_shape = (CH,) + feat
    feat_elems = 1
    for f in feat:
        feat_elems *= f
    nvec = (CH * feat_elems) // 16

    mesh = plsc.VectorSubcoreMesh(core_axis_name="c", subcore_axis_name="s")

    def body(*refs):
        if gather:
            (hs_hbm, zeros_hbm, src_hbm, dst_hbm, out_hbm, src_v, dst_v,
             rows_v, acc_sh, gsem0, gsem1, ssem0, ssem1) = refs
        else:
            zeros_hbm, dst_hbm, out_hbm, dst_v, rows_v, acc_sh = refs
        c = lax.axis_index("c")
        s = lax.axis_index("s")
        on = c == 0

        r0 = s * RPT

        @pl.when(on)
        def _():
            pltpu.sync_copy(zeros_hbm.at[pl.ds(r0, RPT)],
                            acc_sh.at[pl.ds(r0, RPT)])

        plsc.subcore_barrier()

        if gather:
            gsems = [gsem0, gsem1]
            ssems = [ssem0, ssem1]

            def g_start(j, buf):
                pltpu.async_copy(hs_hbm.at[src_v.at[j]], rows_v.at[buf],
                                 gsems[buf])

            def g_wait(j, buf):
                pltpu.make_async_copy(hs_hbm.at[src_v.at[j]],
                                      rows_v.at[buf], gsems[buf]).wait()

            def s_start(j, buf):
                pltpu.async_copy(rows_v.at[buf], acc_sh.at[dst_v.at[j]],
                                 ssems[buf], add=True)

            def s_wait(j, buf):
                pltpu.make_async_copy(rows_v.at[buf], acc_sh.at[dst_v.at[j]],
                                      ssems[buf]).wait()

            def block(b, _):
                cs = s * NCHUNK_W + b * MBLK
                pltpu.sync_copy(src_hbm.at[pl.ds(cs, MBLK)], src_v)
                pltpu.sync_copy(dst_hbm.at[pl.ds(cs, MBLK)], dst_v)
                g_start(0, 0)
                g_start(1, 1)

                def chunk(it, _):
                    j = 2 * it
                    for buf in range(2):
                        g_wait(j + buf, buf)
                        s_start(j + buf, buf)
                    for buf in range(2):
                        s_wait(j + buf, buf)

                        @pl.when(j + buf + 2 < MBLK)
                        def _():
                            g_start(j + buf + 2, buf)

                    return 0

                lax.fori_loop(0, MBLK // 2, chunk, 0)
                return 0

            @pl.when(on)
            def _():
                lax.fori_loop(0, NCHUNK_W // MBLK, block, 0)
        else:
            @pl.when(on)
            def _():
                pltpu.sync_copy(dst_hbm.at[pl.ds(s * NCHUNK_W, NCHUNK_W)],
                                dst_v)

                def fbody(t, _):
                    rows_v[pl.ds(t * 16, 16)] = jnp.full((16,), 1.0, _f32)
                    return 0

                lax.fori_loop(0, nvec, fbody, 0)

                def chunk(j, _):
                    pltpu.sync_copy(rows_v, acc_sh.at[dst_v.at[j]], add=True)
                    return 0

                lax.fori_loop(0, NCHUNK_W, chunk, 0)
        plsc.subcore_barrier()

        @pl.when(on)
        def _():
            pltpu.sync_copy(acc_sh.at[pl.ds(r0, RPT)],
                            out_hbm.at[pl.ds(r0, RPT)])

    scratch = []
    if gather:
        scratch.append(pltpu.VMEM((MBLK, CH), jnp.int32))
    scratch += [
        pltpu.VMEM((MBLK if gather else NCHUNK_W, CH), jnp.int32),
        pltpu.VMEM(((2,) if gather else ()) + rows_shape, _f32),
        pltpu.VMEM_SHARED((N_PAD,) + feat, _f32),
    ]
    if gather:
        scratch += [pltpu.SemaphoreType.DMA] * 4

    kern = pl.kernel(
        body,
        out_type=jax.ShapeDtypeStruct((N_PAD,) + feat, _f32),
        mesh=mesh,
        scratch_types=scratch,
    )
    return kern


_deg_scatter = _make_edge_scatter((), gather=False)
_scatter_h = _make_edge_scatter((H,), gather=True)


def _tc_first(x, w, degp):
    def body(x_ref, w_ref, deg_ref, hs_ref, dis_ref):
        dis = lax.rsqrt(deg_ref[:N, :] + 1.0)
        dis_ref[...] = dis
        hs_ref[...] = dis * jnp.dot(x_ref[...], w_ref[...],
                                    preferred_element_type=_f32)

    return pl.pallas_call(
        body,
        out_shape=(
            jax.ShapeDtypeStruct((N, w.shape[1]), _f32),
            jax.ShapeDtypeStruct((N, 1), _f32),
        ),
    )(x, w, degp)


def _tc_mid(p, hs, dis, b, w):
    def body(p_ref, hs_ref, dis_ref, b_ref, w_ref, out_ref):
        dis = dis_ref[...]
        a = dis * (p_ref[:N, :] + hs_ref[...]) + b_ref[...]
        h = jnp.maximum(a, 0.0)
        out_ref[...] = dis * jnp.dot(h, w_ref[...], preferred_element_type=_f32)

    return pl.pallas_call(
        body,
        out_shape=jax.ShapeDtypeStruct((N, w.shape[1]), _f32),
    )(p, hs, dis, b, w)


def _tc_last(p, hs, dis, b):
    def body(p_ref, hs_ref, dis_ref, b_ref, out_ref):
        a = dis_ref[...] * (p_ref[:N, :C] + hs_ref[:, :C]) + b_ref[...]
        m = jnp.max(a, axis=1, keepdims=True)
        lse = m + jnp.log(jnp.sum(jnp.exp(a - m), axis=1, keepdims=True))
        out_ref[...] = a - lse

    return pl.pallas_call(
        body,
        out_shape=jax.ShapeDtypeStruct((N, C), _f32),
    )(p, hs, dis, b)


@jax.jit
def kernel(x, edge_index, W1, b1, W2, b2, W3, b3, W4, b4):
    pad = E_PAD - E
    padi = jnp.arange(pad, dtype=jnp.int32)
    src = jnp.concatenate(
        [edge_index[0], padi % CH]).reshape(NCHUNK_TOT, CH)
    dst = jnp.concatenate(
        [edge_index[1], N + padi % (N_PAD - N)]).reshape(NCHUNK_TOT, CH)

    z1 = jnp.zeros((N_PAD,), _f32)
    zH = jnp.zeros((N_PAD, H), _f32)

    degp = _deg_scatter(z1, dst).reshape(N_PAD, 1)
    hs1, dis = _tc_first(x, W1, degp)

    p = _scatter_h(hs1, zH, src, dst)
    hs2 = _tc_mid(p, hs1, dis, b1.reshape(1, H), W2)
    p = _scatter_h(hs2, zH, src, dst)
    hs3 = _tc_mid(p, hs2, dis, b2.reshape(1, H), W2)
    p = _scatter_h(hs3, zH, src, dst)
    hs4 = _tc_mid(p, hs3, dis, b2.reshape(1, H), W3)
    p = _scatter_h(hs4, zH, src, dst)
    W4p = jnp.pad(W4, ((0, 0), (0, H - C)))
    hs5 = _tc_mid(p, hs4, dis, b3.reshape(1, H), W4p)
    p = _scatter_h(hs5, zH, src, dst)
    return _tc_last(p, hs5, dis, b4.reshape(1, C))


# speedup vs baseline: 2.1119x; 1.0165x over previous
import functools

import jax
import jax.numpy as jnp
from jax import lax
from jax.experimental import pallas as pl
from jax.experimental.pallas import tpu as pltpu
from jax.experimental.pallas import tpu_sc as plsc

N = 10000
E = 320000
D = 128
H = 128
C = 16

NS = 16
CH = 128
MBLK = 16
NBLK = 10
NCHUNK_TOT = 2560
NCHUNK_W = NCHUNK_TOT // NS
E_PAD = NCHUNK_TOT * CH
N_PAD = 10240
RPT = N_PAD // NS

_f32 = jnp.float32


def _make_edge_scatter(feat, gather):
    rows_shape = (CH,) + feat
    feat_elems = 1
    for f in feat:
        feat_elems *= f
    nvec = (CH * feat_elems) // 16

    mesh = plsc.VectorSubcoreMesh(core_axis_name="c", subcore_axis_name="s")

    def body(*refs):
        if gather:
            (hs_hbm, zeros_hbm, src_hbm, dst_hbm, out_hbm, src_v, dst_v,
             rows_v, acc_sh, gsem0, gsem1, ssem0, ssem1, isem_v) = refs
        else:
            zeros_hbm, dst_hbm, out_hbm, dst_v, rows_v, acc_sh = refs
        c = lax.axis_index("c")
        s = lax.axis_index("s")
        on = c == 0

        r0 = s * RPT

        @pl.when(on)
        def _():
            pltpu.sync_copy(zeros_hbm.at[pl.ds(r0, RPT)],
                            acc_sh.at[pl.ds(r0, RPT)])

        plsc.subcore_barrier()

        if gather:
            gsems = [gsem0, gsem1]
            ssems = [ssem0, ssem1]
            base = s * NCHUNK_W

            def idx_ref(v, j):
                return v.at[(j // MBLK) % 2, lax.rem(j, MBLK)]

            def g_start(j, buf):
                pltpu.async_copy(hs_hbm.at[idx_ref(src_v, j)],
                                 rows_v.at[buf], gsems[buf])

            def g_wait(j, buf):
                pltpu.make_async_copy(hs_hbm.at[idx_ref(src_v, j)],
                                      rows_v.at[buf], gsems[buf]).wait()

            def s_start(j, buf):
                pltpu.async_copy(rows_v.at[buf], acc_sh.at[idx_ref(dst_v, j)],
                                 ssems[buf], add=True)

            def s_wait(j, buf):
                pltpu.make_async_copy(rows_v.at[buf],
                                      acc_sh.at[idx_ref(dst_v, j)],
                                      ssems[buf]).wait()

            def i_start(b):
                sb = b % 2
                pltpu.async_copy(src_hbm.at[pl.ds(base + b * MBLK, MBLK)],
                                 src_v.at[sb], isem_v.at[sb])
                pltpu.async_copy(dst_hbm.at[pl.ds(base + b * MBLK, MBLK)],
                                 dst_v.at[sb], isem_v.at[sb])

            def i_wait(b):
                sb = b % 2
                pltpu.make_async_copy(src_hbm.at[pl.ds(base + b * MBLK, MBLK)],
                                      src_v.at[sb], isem_v.at[sb]).wait()
                pltpu.make_async_copy(dst_hbm.at[pl.ds(base + b * MBLK, MBLK)],
                                      dst_v.at[sb], isem_v.at[sb]).wait()

            @pl.when(on)
            def _():
                i_start(0)
                i_wait(0)
                i_start(1)
                g_start(0, 0)
                g_start(1, 1)

                def chunk(it, _):
                    j0 = 2 * it
                    for buf in range(2):
                        g_wait(j0 + buf, buf)
                        s_start(j0 + buf, buf)

                    nxt = j0 + 2

                    @pl.when(jnp.logical_and(nxt < NCHUNK_W,
                                             lax.rem(nxt, MBLK) == 0))
                    def _():
                        i_wait(nxt // MBLK)

                    for buf in range(2):
                        s_wait(j0 + buf, buf)

                        @pl.when(j0 + buf + 2 < NCHUNK_W)
                        def _():
                            g_start(j0 + buf + 2, buf)

                    j1 = j0 + 1

                    @pl.when(jnp.logical_and(lax.rem(j1, MBLK) == MBLK - 1,
                                             j1 // MBLK + 2 < NBLK))
                    def _():
                        i_start(j1 // MBLK + 2)

                    return 0

                lax.fori_loop(0, NCHUNK_W // 2, chunk, 0)
        else:
            @pl.when(on)
            def _():
                pltpu.sync_copy(dst_hbm.at[pl.ds(s * NCHUNK_W, NCHUNK_W)],
                                dst_v)

                def fbody(t, _):
                    rows_v[pl.ds(t * 16, 16)] = jnp.full((16,), 1.0, _f32)
                    return 0

                lax.fori_loop(0, nvec, fbody, 0)

                def chunk(j, _):
                    pltpu.sync_copy(rows_v, acc_sh.at[dst_v.at[j]], add=True)
                    return 0

                lax.fori_loop(0, NCHUNK_W, chunk, 0)
        plsc.subcore_barrier()

        @pl.when(on)
        def _():
            pltpu.sync_copy(acc_sh.at[pl.ds(r0, RPT)],
                            out_hbm.at[pl.ds(r0, RPT)])

    scratch = []
    if gather:
        scratch.append(pltpu.VMEM((2, MBLK, CH), jnp.int32))
    scratch += [
        pltpu.VMEM((2, MBLK, CH) if gather else (NCHUNK_W, CH),
                   jnp.int32),
        pltpu.VMEM(((2,) if gather else ()) + rows_shape, _f32),
        pltpu.VMEM_SHARED((N_PAD,) + feat, _f32),
    ]
    if gather:
        scratch += [pltpu.SemaphoreType.DMA] * 4
        scratch.append(pltpu.SemaphoreType.DMA((2,)))

    kern = pl.kernel(
        body,
        out_type=jax.ShapeDtypeStruct((N_PAD,) + feat, _f32),
        mesh=mesh,
        scratch_types=scratch,
    )
    return kern


_deg_scatter = _make_edge_scatter((), gather=False)
_scatter_h = _make_edge_scatter((H,), gather=True)


def _tc_first(x, w, degp):
    def body(x_ref, w_ref, deg_ref, hs_ref, dis_ref):
        dis = lax.rsqrt(deg_ref[:N, :] + 1.0)
        dis_ref[...] = dis
        hs_ref[...] = dis * jnp.dot(x_ref[...], w_ref[...],
                                    preferred_element_type=_f32)

    return pl.pallas_call(
        body,
        out_shape=(
            jax.ShapeDtypeStruct((N, w.shape[1]), _f32),
            jax.ShapeDtypeStruct((N, 1), _f32),
        ),
    )(x, w, degp)


def _tc_mid(p, hs, dis, b, w):
    def body(p_ref, hs_ref, dis_ref, b_ref, w_ref, out_ref):
        dis = dis_ref[...]
        a = dis * (p_ref[:N, :] + hs_ref[...]) + b_ref[...]
        h = jnp.maximum(a, 0.0)
        out_ref[...] = dis * jnp.dot(h, w_ref[...], preferred_element_type=_f32)

    return pl.pallas_call(
        body,
        out_shape=jax.ShapeDtypeStruct((N, w.shape[1]), _f32),
    )(p, hs, dis, b, w)


def _tc_last(p, hs, dis, b):
    def body(p_ref, hs_ref, dis_ref, b_ref, out_ref):
        a = dis_ref[...] * (p_ref[:N, :C] + hs_ref[:, :C]) + b_ref[...]
        m = jnp.max(a, axis=1, keepdims=True)
        lse = m + jnp.log(jnp.sum(jnp.exp(a - m), axis=1, keepdims=True))
        out_ref[...] = a - lse

    return pl.pallas_call(
        body,
        out_shape=jax.ShapeDtypeStruct((N, C), _f32),
    )(p, hs, dis, b)


@jax.jit
def kernel(x, edge_index, W1, b1, W2, b2, W3, b3, W4, b4):
    pad = E_PAD - E
    padi = jnp.arange(pad, dtype=jnp.int32)
    src = jnp.concatenate(
        [edge_index[0], padi % CH]).reshape(NCHUNK_TOT, CH)
    dst = jnp.concatenate(
        [edge_index[1], N + padi % (N_PAD - N)]).reshape(NCHUNK_TOT, CH)

    z1 = jnp.zeros((N_PAD,), _f32)
    zH = jnp.zeros((N_PAD, H), _f32)

    degp = _deg_scatter(z1, dst).reshape(N_PAD, 1)
    hs1, dis = _tc_first(x, W1, degp)

    p = _scatter_h(hs1, zH, src, dst)
    hs2 = _tc_mid(p, hs1, dis, b1.reshape(1, H), W2)
    p = _scatter_h(hs2, zH, src, dst)
    hs3 = _tc_mid(p, hs2, dis, b2.reshape(1, H), W2)
    p = _scatter_h(hs3, zH, src, dst)
    hs4 = _tc_mid(p, hs3, dis, b2.reshape(1, H), W3)
    p = _scatter_h(hs4, zH, src, dst)
    W4p = jnp.pad(W4, ((0, 0), (0, H - C)))
    hs5 = _tc_mid(p, hs4, dis, b3.reshape(1, H), W4p)
    p = _scatter_h(hs5, zH, src, dst)
    return _tc_last(p, hs5, dis, b4.reshape(1, C))


# speedup vs baseline: 2.9538x; 1.3987x over previous
import functools

import jax
import jax.numpy as jnp
from jax import lax
from jax.experimental import pallas as pl
from jax.experimental.pallas import tpu as pltpu
from jax.experimental.pallas import tpu_sc as plsc

N = 10000
E = 320000
D = 128
H = 128
C = 16

NS = 16
CH = 96
MBLK = 16
NBLK = 14
NCHUNK_W = MBLK * NBLK
NCHUNK_TOT = NCHUNK_W * NS
E_PAD = NCHUNK_TOT * CH
N_PAD = 10240
RPT = N_PAD // NS

_f32 = jnp.float32


def _make_edge_scatter(feat, gather):
    rows_shape = (CH,) + feat
    feat_elems = 1
    for f in feat:
        feat_elems *= f
    nvec = (CH * feat_elems) // 16

    mesh = plsc.VectorSubcoreMesh(core_axis_name="c", subcore_axis_name="s")

    def body(*refs):
        if gather:
            (hs_hbm, zeros_hbm, src_hbm, dst_hbm, out_hbm, src_v, dst_v,
             rows_v, acc_sh, gsem_v, ssem_v, isem_v) = refs
        else:
            zeros_hbm, dst_hbm, out_hbm, dst_v, rows_v, acc_sh = refs
        c = lax.axis_index("c")
        s = lax.axis_index("s")
        on = c == 0

        r0 = s * RPT

        @pl.when(on)
        def _():
            pltpu.sync_copy(zeros_hbm.at[pl.ds(r0, RPT)],
                            acc_sh.at[pl.ds(r0, RPT)])

        plsc.subcore_barrier()

        if gather:
            base = s * NCHUNK_W

            def idx_ref(v, j):
                return v.at[(j // MBLK) % 2, lax.rem(j, MBLK)]

            def g_start(j):
                pltpu.async_copy(hs_hbm.at[idx_ref(src_v, j)],
                                 rows_v.at[lax.rem(j, 3)],
                                 gsem_v.at[lax.rem(j, 3)])

            def g_wait(j):
                pltpu.make_async_copy(hs_hbm.at[idx_ref(src_v, j)],
                                      rows_v.at[lax.rem(j, 3)],
                                      gsem_v.at[lax.rem(j, 3)]).wait()

            def s_start(j):
                pltpu.async_copy(rows_v.at[lax.rem(j, 3)],
                                 acc_sh.at[idx_ref(dst_v, j)],
                                 ssem_v.at[lax.rem(j, 3)], add=True)

            def s_wait(j):
                pltpu.make_async_copy(rows_v.at[lax.rem(j, 3)],
                                      acc_sh.at[idx_ref(dst_v, j)],
                                      ssem_v.at[lax.rem(j, 3)]).wait()

            def i_start(b):
                sb = b % 2
                pltpu.async_copy(src_hbm.at[pl.ds(base + b * MBLK, MBLK)],
                                 src_v.at[sb], isem_v.at[sb])
                pltpu.async_copy(dst_hbm.at[pl.ds(base + b * MBLK, MBLK)],
                                 dst_v.at[sb], isem_v.at[sb])

            def i_wait(b):
                sb = b % 2
                pltpu.make_async_copy(src_hbm.at[pl.ds(base + b * MBLK, MBLK)],
                                      src_v.at[sb], isem_v.at[sb]).wait()
                pltpu.make_async_copy(dst_hbm.at[pl.ds(base + b * MBLK, MBLK)],
                                      dst_v.at[sb], isem_v.at[sb]).wait()

            @pl.when(on)
            def _():
                i_start(0)
                i_wait(0)
                i_start(1)
                g_start(0)
                g_start(1)

                def chunk(j, _):
                    g_wait(j)
                    s_start(j)

                    @pl.when(jnp.logical_and(
                        jnp.logical_and(lax.rem(j, MBLK) == 1, j > MBLK),
                        j // MBLK + 1 < NBLK))
                    def _():
                        i_start(j // MBLK + 1)

                    @pl.when(j >= 1)
                    def _():
                        s_wait(j - 1)

                    @pl.when(jnp.logical_and(j + 2 < NCHUNK_W,
                                             lax.rem(j + 2, MBLK) == 0))
                    def _():
                        i_wait((j + 2) // MBLK)

                    @pl.when(j + 2 < NCHUNK_W)
                    def _():
                        g_start(j + 2)

                    return 0

                lax.fori_loop(0, NCHUNK_W, chunk, 0)
                s_wait(NCHUNK_W - 1)
        else:
            @pl.when(on)
            def _():
                pltpu.sync_copy(dst_hbm.at[pl.ds(s * NCHUNK_W, NCHUNK_W)],
                                dst_v)

                def fbody(t, _):
                    rows_v[pl.ds(t * 16, 16)] = jnp.full((16,), 1.0, _f32)
                    return 0

                lax.fori_loop(0, nvec, fbody, 0)

                def chunk(j, _):
                    pltpu.sync_copy(rows_v, acc_sh.at[dst_v.at[j]], add=True)
                    return 0

                lax.fori_loop(0, NCHUNK_W, chunk, 0)
        plsc.subcore_barrier()

        @pl.when(on)
        def _():
            pltpu.sync_copy(acc_sh.at[pl.ds(r0, RPT)],
                            out_hbm.at[pl.ds(r0, RPT)])

    scratch = []
    if gather:
        scratch.append(pltpu.VMEM((2, MBLK, CH), jnp.int32))
    scratch += [
        pltpu.VMEM((2, MBLK, CH) if gather else (NCHUNK_W, CH),
                   jnp.int32),
        pltpu.VMEM(((3,) if gather else ()) + rows_shape, _f32),
        pltpu.VMEM_SHARED((N_PAD,) + feat, _f32),
    ]
    if gather:
        scratch += [pltpu.SemaphoreType.DMA((3,)),
                    pltpu.SemaphoreType.DMA((3,)),
                    pltpu.SemaphoreType.DMA((2,))]

    kern = pl.kernel(
        body,
        out_type=jax.ShapeDtypeStruct((N_PAD,) + feat, _f32),
        mesh=mesh,
        scratch_types=scratch,
    )
    return kern


_deg_scatter = _make_edge_scatter((), gather=False)
_scatter_h = _make_edge_scatter((H,), gather=True)


def _tc_first(x, w, degp):
    def body(x_ref, w_ref, deg_ref, hs_ref, dis_ref):
        dis = lax.rsqrt(deg_ref[:N, :] + 1.0)
        dis_ref[...] = dis
        hs_ref[...] = dis * jnp.dot(x_ref[...], w_ref[...],
                                    preferred_element_type=_f32)

    return pl.pallas_call(
        body,
        out_shape=(
            jax.ShapeDtypeStruct((N, w.shape[1]), _f32),
            jax.ShapeDtypeStruct((N, 1), _f32),
        ),
    )(x, w, degp)


def _tc_mid(p, hs, dis, b, w):
    def body(p_ref, hs_ref, dis_ref, b_ref, w_ref, out_ref):
        dis = dis_ref[...]
        a = dis * (p_ref[:N, :] + hs_ref[...]) + b_ref[...]
        h = jnp.maximum(a, 0.0)
        out_ref[...] = dis * jnp.dot(h, w_ref[...], preferred_element_type=_f32)

    return pl.pallas_call(
        body,
        out_shape=jax.ShapeDtypeStruct((N, w.shape[1]), _f32),
    )(p, hs, dis, b, w)


def _tc_last(p, hs, dis, b):
    def body(p_ref, hs_ref, dis_ref, b_ref, out_ref):
        a = dis_ref[...] * (p_ref[:N, :C] + hs_ref[:, :C]) + b_ref[...]
        m = jnp.max(a, axis=1, keepdims=True)
        lse = m + jnp.log(jnp.sum(jnp.exp(a - m), axis=1, keepdims=True))
        out_ref[...] = a - lse

    return pl.pallas_call(
        body,
        out_shape=jax.ShapeDtypeStruct((N, C), _f32),
    )(p, hs, dis, b)


@jax.jit
def kernel(x, edge_index, W1, b1, W2, b2, W3, b3, W4, b4):
    pad = E_PAD - E
    padi = jnp.arange(pad, dtype=jnp.int32)
    src = jnp.concatenate(
        [edge_index[0], padi % CH]).reshape(NCHUNK_TOT, CH)
    dst = jnp.concatenate(
        [edge_index[1], N + padi % (N_PAD - N)]).reshape(NCHUNK_TOT, CH)

    z1 = jnp.zeros((N_PAD,), _f32)
    zH = jnp.zeros((N_PAD, H), _f32)

    degp = _deg_scatter(z1, dst).reshape(N_PAD, 1)
    hs1, dis = _tc_first(x, W1, degp)

    p = _scatter_h(hs1, zH, src, dst)
    hs2 = _tc_mid(p, hs1, dis, b1.reshape(1, H), W2)
    p = _scatter_h(hs2, zH, src, dst)
    hs3 = _tc_mid(p, hs2, dis, b2.reshape(1, H), W2)
    p = _scatter_h(hs3, zH, src, dst)
    hs4 = _tc_mid(p, hs3, dis, b2.reshape(1, H), W3)
    p = _scatter_h(hs4, zH, src, dst)
    W4p = jnp.pad(W4, ((0, 0), (0, H - C)))
    hs5 = _tc_mid(p, hs4, dis, b3.reshape(1, H), W4p)
    p = _scatter_h(hs5, zH, src, dst)
    return _tc_last(p, hs5, dis, b4.reshape(1, C))


# speedup vs baseline: 3.0392x; 1.0289x over previous
import functools

import jax
import jax.numpy as jnp
from jax import lax
from jax.experimental import pallas as pl
from jax.experimental.pallas import tpu as pltpu
from jax.experimental.pallas import tpu_sc as plsc

N = 10000
E = 320000
D = 128
H = 128
C = 16

NS = 16
CH = 112
MBLK = 8
NBLK = 23
NCHUNK_W = MBLK * NBLK
NCHUNK_TOT = NCHUNK_W * NS
E_PAD = NCHUNK_TOT * CH
N_PAD = 10240
RPT = N_PAD // NS

_f32 = jnp.float32


def _make_edge_scatter(feat, gather):
    rows_shape = (CH,) + feat
    feat_elems = 1
    for f in feat:
        feat_elems *= f
    nvec = (CH * feat_elems) // 16

    mesh = plsc.VectorSubcoreMesh(core_axis_name="c", subcore_axis_name="s")

    def body(*refs):
        if gather:
            (hs_hbm, zeros_hbm, src_hbm, dst_hbm, out_hbm, src_v, dst_v,
             rows_v, acc_sh, gsem_v, ssem_v, isem_v) = refs
        else:
            zeros_hbm, dst_hbm, out_hbm, dst_v, rows_v, acc_sh = refs
        c = lax.axis_index("c")
        s = lax.axis_index("s")
        on = c == 0

        r0 = s * RPT

        @pl.when(on)
        def _():
            pltpu.sync_copy(zeros_hbm.at[pl.ds(r0, RPT)],
                            acc_sh.at[pl.ds(r0, RPT)])

        plsc.subcore_barrier()

        if gather:
            base = s * NCHUNK_W

            def idx_ref(v, j):
                return v.at[(j // MBLK) % 2, lax.rem(j, MBLK)]

            def g_start(j):
                pltpu.async_copy(hs_hbm.at[idx_ref(src_v, j)],
                                 rows_v.at[lax.rem(j, 3)],
                                 gsem_v.at[lax.rem(j, 3)])

            def g_wait(j):
                pltpu.make_async_copy(hs_hbm.at[idx_ref(src_v, j)],
                                      rows_v.at[lax.rem(j, 3)],
                                      gsem_v.at[lax.rem(j, 3)]).wait()

            def s_start(j):
                pltpu.async_copy(rows_v.at[lax.rem(j, 3)],
                                 acc_sh.at[idx_ref(dst_v, j)],
                                 ssem_v.at[lax.rem(j, 3)], add=True)

            def s_wait(j):
                pltpu.make_async_copy(rows_v.at[lax.rem(j, 3)],
                                      acc_sh.at[idx_ref(dst_v, j)],
                                      ssem_v.at[lax.rem(j, 3)]).wait()

            def i_start(b):
                sb = b % 2
                pltpu.async_copy(src_hbm.at[pl.ds(base + b * MBLK, MBLK)],
                                 src_v.at[sb], isem_v.at[sb])
                pltpu.async_copy(dst_hbm.at[pl.ds(base + b * MBLK, MBLK)],
                                 dst_v.at[sb], isem_v.at[sb])

            def i_wait(b):
                sb = b % 2
                pltpu.make_async_copy(src_hbm.at[pl.ds(base + b * MBLK, MBLK)],
                                      src_v.at[sb], isem_v.at[sb]).wait()
                pltpu.make_async_copy(dst_hbm.at[pl.ds(base + b * MBLK, MBLK)],
                                      dst_v.at[sb], isem_v.at[sb]).wait()

            @pl.when(on)
            def _():
                i_start(0)
                i_wait(0)
                i_start(1)
                g_start(0)
                g_start(1)

                def chunk(j, _):
                    g_wait(j)
                    s_start(j)

                    @pl.when(jnp.logical_and(
                        jnp.logical_and(lax.rem(j, MBLK) == 1, j > MBLK),
                        j // MBLK + 1 < NBLK))
                    def _():
                        i_start(j // MBLK + 1)

                    @pl.when(j >= 1)
                    def _():
                        s_wait(j - 1)

                    @pl.when(jnp.logical_and(j + 2 < NCHUNK_W,
                                             lax.rem(j + 2, MBLK) == 0))
                    def _():
                        i_wait((j + 2) // MBLK)

                    @pl.when(j + 2 < NCHUNK_W)
                    def _():
                        g_start(j + 2)

                    return 0

                lax.fori_loop(0, NCHUNK_W, chunk, 0)
                s_wait(NCHUNK_W - 1)
        else:
            @pl.when(on)
            def _():
                pltpu.sync_copy(dst_hbm.at[pl.ds(s * NCHUNK_W, NCHUNK_W)],
                                dst_v)

                def fbody(t, _):
                    rows_v[pl.ds(t * 16, 16)] = jnp.full((16,), 1.0, _f32)
                    return 0

                lax.fori_loop(0, nvec, fbody, 0)

                def chunk(j, _):
                    pltpu.sync_copy(rows_v, acc_sh.at[dst_v.at[j]], add=True)
                    return 0

                lax.fori_loop(0, NCHUNK_W, chunk, 0)
        plsc.subcore_barrier()

        @pl.when(on)
        def _():
            pltpu.sync_copy(acc_sh.at[pl.ds(r0, RPT)],
                            out_hbm.at[pl.ds(r0, RPT)])

    scratch = []
    if gather:
        scratch.append(pltpu.VMEM((2, MBLK, CH), jnp.int32))
    scratch += [
        pltpu.VMEM((2, MBLK, CH) if gather else (NCHUNK_W, CH),
                   jnp.int32),
        pltpu.VMEM(((3,) if gather else ()) + rows_shape, _f32),
        pltpu.VMEM_SHARED((N_PAD,) + feat, _f32),
    ]
    if gather:
        scratch += [pltpu.SemaphoreType.DMA((3,)),
                    pltpu.SemaphoreType.DMA((3,)),
                    pltpu.SemaphoreType.DMA((2,))]

    kern = pl.kernel(
        body,
        out_type=jax.ShapeDtypeStruct((N_PAD,) + feat, _f32),
        mesh=mesh,
        scratch_types=scratch,
    )
    return kern


_deg_scatter = _make_edge_scatter((), gather=False)
_scatter_h = _make_edge_scatter((H,), gather=True)


def _tc_first(x, w, degp):
    def body(x_ref, w_ref, deg_ref, hs_ref, dis_ref):
        dis = lax.rsqrt(deg_ref[:N, :] + 1.0)
        dis_ref[...] = dis
        hs_ref[...] = dis * jnp.dot(x_ref[...], w_ref[...],
                                    preferred_element_type=_f32)

    return pl.pallas_call(
        body,
        out_shape=(
            jax.ShapeDtypeStruct((N, w.shape[1]), _f32),
            jax.ShapeDtypeStruct((N, 1), _f32),
        ),
    )(x, w, degp)


def _tc_mid(p, hs, dis, b, w):
    def body(p_ref, hs_ref, dis_ref, b_ref, w_ref, out_ref):
        dis = dis_ref[...]
        a = dis * (p_ref[:N, :] + hs_ref[...]) + b_ref[...]
        h = jnp.maximum(a, 0.0)
        out_ref[...] = dis * jnp.dot(h, w_ref[...], preferred_element_type=_f32)

    return pl.pallas_call(
        body,
        out_shape=jax.ShapeDtypeStruct((N, w.shape[1]), _f32),
    )(p, hs, dis, b, w)


def _tc_last(p, hs, dis, b):
    def body(p_ref, hs_ref, dis_ref, b_ref, out_ref):
        a = dis_ref[...] * (p_ref[:N, :C] + hs_ref[:, :C]) + b_ref[...]
        m = jnp.max(a, axis=1, keepdims=True)
        lse = m + jnp.log(jnp.sum(jnp.exp(a - m), axis=1, keepdims=True))
        out_ref[...] = a - lse

    return pl.pallas_call(
        body,
        out_shape=jax.ShapeDtypeStruct((N, C), _f32),
    )(p, hs, dis, b)


@jax.jit
def kernel(x, edge_index, W1, b1, W2, b2, W3, b3, W4, b4):
    pad = E_PAD - E
    padi = jnp.arange(pad, dtype=jnp.int32)
    src = jnp.concatenate(
        [edge_index[0], padi % CH]).reshape(NCHUNK_TOT, CH)
    dst = jnp.concatenate(
        [edge_index[1], N + padi % (N_PAD - N)]).reshape(NCHUNK_TOT, CH)

    z1 = jnp.zeros((N_PAD,), _f32)
    zH = jnp.zeros((N_PAD, H), _f32)

    degp = _deg_scatter(z1, dst).reshape(N_PAD, 1)
    hs1, dis = _tc_first(x, W1, degp)

    p = _scatter_h(hs1, zH, src, dst)
    hs2 = _tc_mid(p, hs1, dis, b1.reshape(1, H), W2)
    p = _scatter_h(hs2, zH, src, dst)
    hs3 = _tc_mid(p, hs2, dis, b2.reshape(1, H), W2)
    p = _scatter_h(hs3, zH, src, dst)
    hs4 = _tc_mid(p, hs3, dis, b2.reshape(1, H), W3)
    p = _scatter_h(hs4, zH, src, dst)
    W4p = jnp.pad(W4, ((0, 0), (0, H - C)))
    hs5 = _tc_mid(p, hs4, dis, b3.reshape(1, H), W4p)
    p = _scatter_h(hs5, zH, src, dst)
    return _tc_last(p, hs5, dis, b4.reshape(1, C))
